# speedup vs baseline: 1.0180x; 1.0180x over previous
import functools

import jax
import jax.numpy as jnp
from jax import lax
from jax.experimental import pallas as pl
from jax.experimental.pallas import tpu as pltpu
from jax.experimental.pallas import tpu_sc as plsc

VOCAB = 100000
EMB = 768
SEQ = 50
HID = 768
BATCH = 1024
LN_EPS = 1e-12

N_TOK = BATCH * SEQ

NUM_CORES = 2
NUM_SUBCORES = 16
NW = NUM_CORES * NUM_SUBCORES
PER_W = N_TOK // NW
CHUNK = 64
N_CHUNKS = PER_W // CHUNK


def _sc_gather_body(idx_hbm, table_hbm, out_hbm, idx_v, rows_v, sem):
    wid = lax.axis_index("s") * NUM_CORES + lax.axis_index("c")
    base = wid * PER_W

    def body(j, carry):
        off = base + j * CHUNK
        pltpu.sync_copy(idx_hbm.at[pl.ds(off, CHUNK)], idx_v)
        pltpu.async_copy(table_hbm.at[idx_v], rows_v, sem).wait()
        pltpu.sync_copy(rows_v, out_hbm.at[pl.ds(off, CHUNK)])
        return carry

    lax.fori_loop(0, N_CHUNKS, body, 0)


_sc_gather = functools.partial(
    pl.kernel,
    out_type=jax.ShapeDtypeStruct((N_TOK, EMB), jnp.float32),
    mesh=plsc.VectorSubcoreMesh(
        core_axis_name="c", subcore_axis_name="s",
        num_cores=NUM_CORES, num_subcores=NUM_SUBCORES),
    scratch_types=[
        pltpu.VMEM((CHUNK,), jnp.int32),
        pltpu.VMEM((CHUNK, EMB), jnp.float32),
        pltpu.SemaphoreType.DMA,
    ],
)(_sc_gather_body)


BT = 800
GRID = N_TOK // BT


def _tc_body(g_ref, pos_ref, ls_ref, lb_ref, w_ref, b_ref, o_ref):
    h = g_ref[...] + pos_ref[...]
    m = jnp.mean(h, axis=1, keepdims=True)
    hc = h - m
    v = jnp.mean(hc * hc, axis=1, keepdims=True)
    hn = hc * lax.rsqrt(v + LN_EPS)
    hn = hn * ls_ref[...] + lb_ref[...]
    o_ref[...] = (
        jnp.dot(hn, w_ref[...], preferred_element_type=jnp.float32)
        + b_ref[...]
    )


def _tc_call(gathered, pos_tiled, ln_scale, ln_bias, dense_kernel, dense_bias):
    return pl.pallas_call(
        _tc_body,
        grid=(GRID,),
        in_specs=[
            pl.BlockSpec((BT, EMB), lambda i: (i, 0)),
            pl.BlockSpec((BT, EMB), lambda i: (0, 0)),
            pl.BlockSpec((1, EMB), lambda i: (0, 0)),
            pl.BlockSpec((1, EMB), lambda i: (0, 0)),
            pl.BlockSpec((EMB, HID), lambda i: (0, 0)),
            pl.BlockSpec((1, HID), lambda i: (0, 0)),
        ],
        out_specs=pl.BlockSpec((BT, HID), lambda i: (i, 0)),
        out_shape=jax.ShapeDtypeStruct((N_TOK, HID), jnp.float32),
    )(gathered, pos_tiled, ln_scale, ln_bias, dense_kernel, dense_bias)


def kernel(x, word_emb, pos_emb, ln_scale, ln_bias, dense_kernel, dense_bias):
    idx = x.reshape(-1).astype(jnp.int32)
    gathered = _sc_gather(idx, word_emb)
    pos_tiled = jnp.tile(pos_emb, (BT // SEQ, 1))
    out = _tc_call(
        gathered,
        pos_tiled,
        ln_scale.reshape(1, EMB),
        ln_bias.reshape(1, EMB),
        dense_kernel,
        dense_bias.reshape(1, HID),
    )
    return out.reshape(BATCH, SEQ, HID)


# speedup vs baseline: 1.9778x; 1.9429x over previous
import functools

import jax
import jax.numpy as jnp
from jax import lax
from jax.experimental import pallas as pl
from jax.experimental.pallas import tpu as pltpu
from jax.experimental.pallas import tpu_sc as plsc

VOCAB = 100000
EMB = 768
SEQ = 50
HID = 768
BATCH = 1024
LN_EPS = 1e-12

N_TOK = BATCH * SEQ

NUM_CORES = 2
NUM_SUBCORES = 16
NW = NUM_CORES * NUM_SUBCORES
PER_W = N_TOK // NW
CHUNK = 64
N_CHUNKS = PER_W // CHUNK


def _sc_gather_body(idx_hbm, table_hbm, out_hbm, idx_v, rows_v, sem):
    wid = lax.axis_index("s") * NUM_CORES + lax.axis_index("c")
    base = wid * PER_W

    def body(j, carry):
        off = base + j * CHUNK
        pltpu.sync_copy(idx_hbm.at[pl.ds(off, CHUNK)], idx_v)
        pltpu.async_copy(table_hbm.at[idx_v], rows_v, sem).wait()
        pltpu.sync_copy(rows_v, out_hbm.at[pl.ds(off, CHUNK)])
        return carry

    lax.fori_loop(0, N_CHUNKS, body, 0)


_sc_gather = functools.partial(
    pl.kernel,
    out_type=jax.ShapeDtypeStruct((N_TOK, EMB), jnp.float32),
    mesh=plsc.VectorSubcoreMesh(
        core_axis_name="c", subcore_axis_name="s",
        num_cores=NUM_CORES, num_subcores=NUM_SUBCORES),
    scratch_types=[
        pltpu.VMEM((CHUNK,), jnp.int32),
        pltpu.VMEM((CHUNK, EMB), jnp.float32),
        pltpu.SemaphoreType.DMA,
    ],
)(_sc_gather_body)


def _tc_body(g_ref, pos_ref, ls_ref, lb_ref, w_ref, b_ref, o_ref):
    h = g_ref[...] + pos_ref[0]
    m = jnp.mean(h, axis=1, keepdims=True)
    hc = h - m
    v = jnp.mean(hc * hc, axis=1, keepdims=True)
    hn = hc * lax.rsqrt(v + LN_EPS)
    hn = hn * ls_ref[...] + lb_ref[...]
    o_ref[0] = (
        jnp.dot(hn, w_ref[...], preferred_element_type=jnp.float32)
        + b_ref[...]
    )


def _tc_call(gathered, pos_emb, ln_scale, ln_bias, dense_kernel, dense_bias):
    return pl.pallas_call(
        _tc_body,
        grid=(SEQ,),
        in_specs=[
            pl.BlockSpec((BATCH, EMB), lambda i: (i, 0)),
            pl.BlockSpec((1, 1, EMB), lambda i: (i, 0, 0)),
            pl.BlockSpec((1, EMB), lambda i: (0, 0)),
            pl.BlockSpec((1, EMB), lambda i: (0, 0)),
            pl.BlockSpec((EMB, HID), lambda i: (0, 0)),
            pl.BlockSpec((1, HID), lambda i: (0, 0)),
        ],
        out_specs=pl.BlockSpec((1, BATCH, HID), lambda i: (i, 0, 0)),
        out_shape=jax.ShapeDtypeStruct((SEQ, BATCH, HID), jnp.float32),
    )(gathered, pos_emb, ln_scale, ln_bias, dense_kernel, dense_bias)


def kernel(x, word_emb, pos_emb, ln_scale, ln_bias, dense_kernel, dense_bias):
    idx = x.T.reshape(-1).astype(jnp.int32)
    gathered = _sc_gather(idx, word_emb)
    out_t = _tc_call(
        gathered,
        pos_emb.reshape(SEQ, 1, EMB),
        ln_scale.reshape(1, EMB),
        ln_bias.reshape(1, EMB),
        dense_kernel,
        dense_bias.reshape(1, HID),
    )
    return out_t.transpose(1, 0, 2)
